# baseline (speedup 1.0000x reference)
import jax
import jax.numpy as jnp
from jax import lax
from jax.experimental import pallas as pl
from jax.experimental.pallas import tpu as pltpu
from jax.experimental.pallas import tpu_sc as plsc

NUM_EMB = 1_000_000
D = 32
BATCH = 16384
HIST = 50
NC = 2
NS = 16
NW = NC * NS
BCHUNK = 1024
NBC = BATCH // BCHUNK
UNITS = HIST * NBC
PER_W = UNITS // NW
SUB = 128
GROUP = BCHUNK // SUB
TRS = 1032


def _body(xt_hbm, w_hbm, out_hbm, idx0, idx1, rows0, rows1, tr_v,
          gsem0, gsem1, wsem):
    wid = lax.axis_index("s") * NC + lax.axis_index("c")
    base_u = wid * PER_W
    lane8 = lax.broadcasted_iota(jnp.int32, (16,), 0) * 8
    idx_b = (idx0, idx1)
    rows_b = (rows0, rows1)
    gsem_b = (gsem0, gsem1)

    def load_and_fire(u, b):
        g = base_u + u
        pltpu.sync_copy(
            xt_hbm.at[g // NBC, pl.ds((g % NBC) * GROUP, GROUP)], idx_b[b]
        )
        return [
            pltpu.async_copy(
                w_hbm.at[idx_b[b].at[j]],
                rows_b[b].at[pl.ds(j * SUB, SUB)],
                gsem_b[b],
            )
            for j in range(GROUP)
        ]

    def drain_gathers(u, b):
        g = base_u + u
        for j in range(GROUP):
            pltpu.make_async_copy(
                w_hbm.at[idx_b[b].at[j]],
                rows_b[b].at[pl.ds(j * SUB, SUB)],
                gsem_b[b],
            ).wait()

    def wb_descs(u):
        g = base_u + u
        h = g // NBC
        bt0 = (g % NBC) * (BCHUNK // SUB)
        return [
            (
                tr_v.at[pl.ds(f * 8, 8), pl.ds(0, SUB)],
                out_hbm.at[
                    h * 4 + f // 8,
                    pl.ds(bt0, 8),
                    pl.ds((f % 8) * SUB, SUB),
                ],
            )
            for f in range(D)
        ]

    def transpose(b):
        def row_block(r, c):
            for k in range(4):
                i = r * 4 + k
                iq = i // SUB
                cols = jnp.full((16,), i % SUB, jnp.int32)
                lo = rows_b[b][i, pl.ds(0, 16)]
                hi = rows_b[b][i, pl.ds(16, 16)]
                plsc.store_scatter(tr_v, [lane8 + iq, cols], lo)
                plsc.store_scatter(tr_v, [lane8 + (128 + iq), cols], hi)
            return c

        lax.fori_loop(0, BCHUNK // 4, row_block, 0)

    def substep(u, b, fire_next):
        if fire_next:
            load_and_fire(u + 1, 1 - b)
        drain_gathers(u, b)
        for src, dst in wb_descs(u - 1):
            pltpu.make_async_copy(src, dst, wsem).wait()
        transpose(b)
        for src, dst in wb_descs(u):
            pltpu.async_copy(src, dst, wsem)

    load_and_fire(0, 0)
    load_and_fire(1, 1)
    drain_gathers(0, 0)
    transpose(0)
    for src, dst in wb_descs(0):
        pltpu.async_copy(src, dst, wsem)

    def pair(i, carry):
        substep(2 * i + 1, 1, True)
        substep(2 * i + 2, 0, True)
        return carry

    lax.fori_loop(0, (PER_W - 3) // 2, pair, 0)
    substep(PER_W - 2, 1, True)
    substep(PER_W - 1, 0, False)
    for src, dst in wb_descs(PER_W - 1):
        pltpu.make_async_copy(src, dst, wsem).wait()


def kernel(x, W):
    xt = x.T.reshape(HIST, NBC * GROUP, SUB)
    mesh = plsc.VectorSubcoreMesh(core_axis_name="c", subcore_axis_name="s")
    out = pl.kernel(
        _body,
        out_type=jax.ShapeDtypeStruct((HIST * 4, SUB, 1024), jnp.float32),
        mesh=mesh,
        compiler_params=pltpu.CompilerParams(
            use_tc_tiling_on_sc=False, needs_layout_passes=False
        ),
        scratch_types=[
            pltpu.VMEM((GROUP, SUB), jnp.int32),
            pltpu.VMEM((GROUP, SUB), jnp.int32),
            pltpu.VMEM((BCHUNK, D), jnp.float32),
            pltpu.VMEM((BCHUNK, D), jnp.float32),
            pltpu.VMEM((D * 8, 129), jnp.float32),
            pltpu.SemaphoreType.DMA,
            pltpu.SemaphoreType.DMA,
            pltpu.SemaphoreType.DMA,
        ],
    )(xt, W)
    out5 = out.reshape(HIST, 4, SUB, 8, SUB)
    return out5.transpose(2, 4, 0, 1, 3).reshape(BATCH, HIST, D)


# speedup vs baseline: 1.3649x; 1.3649x over previous
import jax
import jax.numpy as jnp
from jax import lax
from jax.experimental import pallas as pl
from jax.experimental.pallas import tpu as pltpu
from jax.experimental.pallas import tpu_sc as plsc

NUM_EMB = 1_000_000
D = 32
BATCH = 16384
HIST = 50
NC = 2
NS = 16
NW = NC * NS
BCHUNK = 1024
NBC = BATCH // BCHUNK
UNITS = HIST * NBC
PER_W = UNITS // NW
SUB = 128
GROUP = BCHUNK // SUB
TRS = 1032


def _body(xt_hbm, w_hbm, out_hbm, idx0, idx1, rows0, rows1, tr_v,
          gsem0, gsem1, wsem):
    wid = lax.axis_index("s") * NC + lax.axis_index("c")
    base_u = wid * PER_W
    lane_trs = lax.broadcasted_iota(jnp.int32, (16,), 0) * TRS
    idx_b = (idx0, idx1)
    rows_b = (rows0, rows1)
    gsem_b = (gsem0, gsem1)

    def load_and_fire(u, b):
        g = base_u + u
        pltpu.sync_copy(
            xt_hbm.at[g // NBC, pl.ds((g % NBC) * GROUP, GROUP)], idx_b[b]
        )
        return [
            pltpu.async_copy(
                w_hbm.at[idx_b[b].at[j]],
                rows_b[b].at[pl.ds(j * SUB, SUB)],
                gsem_b[b],
            )
            for j in range(GROUP)
        ]

    def drain_gathers(u, b):
        g = base_u + u
        for j in range(GROUP):
            pltpu.make_async_copy(
                w_hbm.at[idx_b[b].at[j]],
                rows_b[b].at[pl.ds(j * SUB, SUB)],
                gsem_b[b],
            ).wait()

    def wb_descs(u):
        g = base_u + u
        out_base = (g // NBC) * D * BATCH + (g % NBC) * BCHUNK
        return [
            (
                tr_v.at[pl.ds(f * TRS, BCHUNK)],
                out_hbm.at[pl.ds(out_base + f * BATCH, BCHUNK)],
            )
            for f in range(D)
        ]

    def transpose(b):
        def row_block(r, c):
            for k in range(4):
                i = r * 4 + k
                lo = rows_b[b][i, pl.ds(0, 16)]
                hi = rows_b[b][i, pl.ds(16, 16)]
                plsc.store_scatter(tr_v, [lane_trs + i], lo)
                plsc.store_scatter(tr_v, [lane_trs + (16 * TRS + i)], hi)
            return c

        lax.fori_loop(0, BCHUNK // 4, row_block, 0)

    def substep(u, b, fire_next):
        if fire_next:
            load_and_fire(u + 1, 1 - b)
        drain_gathers(u, b)
        for src, dst in wb_descs(u - 1):
            pltpu.make_async_copy(src, dst, wsem).wait()
        transpose(b)
        for src, dst in wb_descs(u):
            pltpu.async_copy(src, dst, wsem)

    load_and_fire(0, 0)
    load_and_fire(1, 1)
    drain_gathers(0, 0)
    transpose(0)
    for src, dst in wb_descs(0):
        pltpu.async_copy(src, dst, wsem)

    def pair(i, carry):
        substep(2 * i + 1, 1, True)
        substep(2 * i + 2, 0, True)
        return carry

    lax.fori_loop(0, (PER_W - 3) // 2, pair, 0)
    substep(PER_W - 2, 1, True)
    substep(PER_W - 1, 0, False)
    for src, dst in wb_descs(PER_W - 1):
        pltpu.make_async_copy(src, dst, wsem).wait()


def kernel(x, W):
    xt = x.T.reshape(HIST, NBC * GROUP, SUB)
    mesh = plsc.VectorSubcoreMesh(core_axis_name="c", subcore_axis_name="s")
    out = pl.kernel(
        _body,
        out_type=jax.ShapeDtypeStruct((HIST * D * BATCH,), jnp.float32),
        mesh=mesh,
        compiler_params=pltpu.CompilerParams(
            use_tc_tiling_on_sc=False, needs_layout_passes=False
        ),
        scratch_types=[
            pltpu.VMEM((GROUP, SUB), jnp.int32),
            pltpu.VMEM((GROUP, SUB), jnp.int32),
            pltpu.VMEM((BCHUNK, D), jnp.float32),
            pltpu.VMEM((BCHUNK, D), jnp.float32),
            pltpu.VMEM((TRS * D,), jnp.float32),
            pltpu.SemaphoreType.DMA,
            pltpu.SemaphoreType.DMA,
            pltpu.SemaphoreType.DMA,
        ],
    )(xt, W)
    return out.reshape(HIST, D, BATCH).transpose(2, 0, 1)
